# baseline (speedup 1.0000x reference)
import functools

import jax
import jax.numpy as jnp
from jax import lax
from jax.experimental import pallas as pl
from jax.experimental.pallas import tpu as pltpu
from jax.experimental.pallas import tpu_sc as plsc

_M, _N = 36864, 384
_M_TAIL = 12288
_M_HEAD = _M - _M_TAIL

_NW = 32
_ROWS_W = _M_TAIL // _NW
_NBUF = 4
_CROWS = 16
_NCHUNK = _ROWS_W // _CROWS
_LANES = 16
_GROUPS = _N // _LANES

_mesh = plsc.VectorSubcoreMesh(core_axis_name="c", subcore_axis_name="s")

_scratch = (
    [pltpu.VMEM((_CROWS, _N), jnp.float32) for _ in range(3 * _NBUF)]
    + [pltpu.SemaphoreType.DMA for _ in range(3 * _NBUF)]
)


@functools.partial(
    pl.kernel,
    out_type=jax.ShapeDtypeStruct((_M_TAIL, _N), jnp.float32),
    mesh=_mesh,
    scratch_types=_scratch,
    compiler_params=pltpu.CompilerParams(use_tc_tiling_on_sc=True),
)
def _sc_add(l_hbm, r_hbm, o_hbm, *refs):
    lbuf = refs[0:_NBUF]
    rbuf = refs[_NBUF:2 * _NBUF]
    obuf = refs[2 * _NBUF:3 * _NBUF]
    sems = refs[3 * _NBUF:]
    lsem = sems[0:_NBUF]
    rsem = sems[_NBUF:2 * _NBUF]
    osem = sems[2 * _NBUF:3 * _NBUF]

    wid = lax.axis_index("s") * 2 + lax.axis_index("c")
    irow = _M_HEAD + wid * _ROWS_W
    orow = wid * _ROWS_W

    def islice(ci):
        return pl.ds(irow + ci * _CROWS, _CROWS)

    def oslice(ci):
        return pl.ds(orow + ci * _CROWS, _CROWS)

    for p in range(_NBUF - 1):
        pltpu.async_copy(l_hbm.at[islice(p)], lbuf[p], lsem[p])
        pltpu.async_copy(r_hbm.at[islice(p)], rbuf[p], rsem[p])

    @pl.loop(0, _NCHUNK, step=_NBUF)
    def chunk_group(ci0):
        for b in range(_NBUF):
            ci = ci0 + b
            pb = (b + _NBUF - 1) % _NBUF

            @pl.when(ci + _NBUF - 1 < _NCHUNK)
            def _start_ahead():
                sl = islice(ci + _NBUF - 1)
                pltpu.async_copy(l_hbm.at[sl], lbuf[pb], lsem[pb])
                pltpu.async_copy(r_hbm.at[sl], rbuf[pb], rsem[pb])

            pltpu.make_async_copy(l_hbm.at[islice(ci)], lbuf[b], lsem[b]).wait()
            pltpu.make_async_copy(r_hbm.at[islice(ci)], rbuf[b], rsem[b]).wait()

            @pl.when(ci >= _NBUF)
            def _drain_prev_out():
                pltpu.make_async_copy(
                    obuf[b], o_hbm.at[oslice(ci)], osem[b]).wait()

            lb, rb_, ob = lbuf[b], rbuf[b], obuf[b]

            def vbody(r):
                for g in range(_GROUPS):
                    sl = pl.ds(g * _LANES, _LANES)
                    ob[r, sl] = lb[r, sl] + rb_[r, sl]

            plsc.parallel_loop(0, _CROWS, 1, unroll=2)(vbody)

            pltpu.async_copy(obuf[b], o_hbm.at[oslice(ci)], osem[b])

    for b in range(_NBUF):
        pltpu.make_async_copy(obuf[b], o_hbm.at[oslice(b)], osem[b]).wait()


_BM = 4096


def _tc_add_body(l_ref, r_ref, o_ref):
    o_ref[...] = l_ref[...] + r_ref[...]


def _tc_head(left, right):
    return pl.pallas_call(
        _tc_add_body,
        grid=(_M_HEAD // _BM,),
        in_specs=[
            pl.BlockSpec((_BM, _N), lambda i: (i, 0)),
            pl.BlockSpec((_BM, _N), lambda i: (i, 0)),
        ],
        out_specs=pl.BlockSpec((_BM, _N), lambda i: (i, 0)),
        out_shape=jax.ShapeDtypeStruct((_M, _N), jnp.float32),
    )(left, right)


def _merge_body(full_ref, tail_ref, o_ref):
    o_ref[...] = tail_ref[...]


def _merge(full, sc_tail):
    nh = _M_HEAD // _BM
    return pl.pallas_call(
        _merge_body,
        grid=(_M_TAIL // _BM,),
        in_specs=[
            pl.BlockSpec((_BM, _N), lambda i, nh=nh: (i + nh, 0)),
            pl.BlockSpec((_BM, _N), lambda i: (i, 0)),
        ],
        out_specs=pl.BlockSpec((_BM, _N), lambda i, nh=nh: (i + nh, 0)),
        out_shape=jax.ShapeDtypeStruct((_M, _N), jnp.float32),
        input_output_aliases={0: 0},
    )(full, sc_tail)


def kernel(left, right):
    sc_tail = _sc_add(left, right)
    full = _tc_head(left, right)
    return _merge(full, sc_tail)


# speedup vs baseline: 1.1188x; 1.1188x over previous
import functools

import jax
import jax.numpy as jnp
from jax import lax
from jax.experimental import pallas as pl
from jax.experimental.pallas import tpu as pltpu
from jax.experimental.pallas import tpu_sc as plsc

_M, _N = 36864, 384
_M_TAIL = 36864
_M_HEAD = _M - _M_TAIL

_NW = 32
_ROWS_W = _M_TAIL // _NW
_NBUF = 6
_CROWS = 16
_NCHUNK = _ROWS_W // _CROWS
_LANES = 16
_GROUPS = _N // _LANES

_mesh = plsc.VectorSubcoreMesh(core_axis_name="c", subcore_axis_name="s")

_scratch = (
    [pltpu.VMEM((_CROWS, _N), jnp.float32) for _ in range(3 * _NBUF)]
    + [pltpu.SemaphoreType.DMA for _ in range(3 * _NBUF)]
)


@functools.partial(
    pl.kernel,
    out_type=jax.ShapeDtypeStruct((_M_TAIL, _N), jnp.float32),
    mesh=_mesh,
    scratch_types=_scratch,
    compiler_params=pltpu.CompilerParams(use_tc_tiling_on_sc=True),
)
def _sc_add(l_hbm, r_hbm, o_hbm, *refs):
    lbuf = refs[0:_NBUF]
    rbuf = refs[_NBUF:2 * _NBUF]
    obuf = refs[2 * _NBUF:3 * _NBUF]
    sems = refs[3 * _NBUF:]
    lsem = sems[0:_NBUF]
    rsem = sems[_NBUF:2 * _NBUF]
    osem = sems[2 * _NBUF:3 * _NBUF]

    wid = lax.axis_index("s") * 2 + lax.axis_index("c")
    irow = _M_HEAD + wid * _ROWS_W
    orow = wid * _ROWS_W

    def islice(ci):
        return pl.ds(irow + ci * _CROWS, _CROWS)

    def oslice(ci):
        return pl.ds(orow + ci * _CROWS, _CROWS)

    for p in range(_NBUF - 1):
        pltpu.async_copy(l_hbm.at[islice(p)], lbuf[p], lsem[p])
        pltpu.async_copy(r_hbm.at[islice(p)], rbuf[p], rsem[p])

    @pl.loop(0, _NCHUNK, step=_NBUF)
    def chunk_group(ci0):
        for b in range(_NBUF):
            ci = ci0 + b
            pb = (b + _NBUF - 1) % _NBUF

            @pl.when(ci + _NBUF - 1 < _NCHUNK)
            def _start_ahead():
                sl = islice(ci + _NBUF - 1)
                pltpu.async_copy(l_hbm.at[sl], lbuf[pb], lsem[pb])
                pltpu.async_copy(r_hbm.at[sl], rbuf[pb], rsem[pb])

            pltpu.make_async_copy(l_hbm.at[islice(ci)], lbuf[b], lsem[b]).wait()
            pltpu.make_async_copy(r_hbm.at[islice(ci)], rbuf[b], rsem[b]).wait()

            @pl.when(ci >= _NBUF)
            def _drain_prev_out():
                pltpu.make_async_copy(
                    obuf[b], o_hbm.at[oslice(ci)], osem[b]).wait()

            lb, rb_, ob = lbuf[b], rbuf[b], obuf[b]

            def vbody(r):
                for g in range(_GROUPS):
                    sl = pl.ds(g * _LANES, _LANES)
                    ob[r, sl] = lb[r, sl] + rb_[r, sl]

            plsc.parallel_loop(0, _CROWS, 1, unroll=2)(vbody)

            pltpu.async_copy(obuf[b], o_hbm.at[oslice(ci)], osem[b])

    for b in range(_NBUF):
        pltpu.make_async_copy(obuf[b], o_hbm.at[oslice(b)], osem[b]).wait()


_BM = 4096


def _tc_add_body(l_ref, r_ref, o_ref):
    o_ref[...] = l_ref[...] + r_ref[...]


def _tc_head(left, right):
    return pl.pallas_call(
        _tc_add_body,
        grid=(_M_HEAD // _BM,),
        in_specs=[
            pl.BlockSpec((_BM, _N), lambda i: (i, 0)),
            pl.BlockSpec((_BM, _N), lambda i: (i, 0)),
        ],
        out_specs=pl.BlockSpec((_BM, _N), lambda i: (i, 0)),
        out_shape=jax.ShapeDtypeStruct((_M, _N), jnp.float32),
    )(left, right)


def _merge_body(full_ref, tail_ref, o_ref):
    o_ref[...] = tail_ref[...]


def _merge(full, sc_tail):
    nh = _M_HEAD // _BM
    return pl.pallas_call(
        _merge_body,
        grid=(_M_TAIL // _BM,),
        in_specs=[
            pl.BlockSpec((_BM, _N), lambda i, nh=nh: (i + nh, 0)),
            pl.BlockSpec((_BM, _N), lambda i: (i, 0)),
        ],
        out_specs=pl.BlockSpec((_BM, _N), lambda i, nh=nh: (i + nh, 0)),
        out_shape=jax.ShapeDtypeStruct((_M, _N), jnp.float32),
        input_output_aliases={0: 0},
    )(full, sc_tail)


def kernel(left, right):
    return _sc_add(left, right)


# speedup vs baseline: 1.1290x; 1.0091x over previous
import functools

import jax
import jax.numpy as jnp
from jax import lax
from jax.experimental import pallas as pl
from jax.experimental.pallas import tpu as pltpu
from jax.experimental.pallas import tpu_sc as plsc

_M, _N = 36864, 384
_M_TAIL = 36864
_M_HEAD = _M - _M_TAIL

_NW = 32
_ROWS_W = _M_TAIL // _NW
_NBUF = 3
_CROWS = 32
_NCHUNK = _ROWS_W // _CROWS
_LANES = 16
_GROUPS = _N // _LANES

_mesh = plsc.VectorSubcoreMesh(core_axis_name="c", subcore_axis_name="s")

_scratch = (
    [pltpu.VMEM((_CROWS, _N), jnp.float32) for _ in range(3 * _NBUF)]
    + [pltpu.SemaphoreType.DMA for _ in range(3 * _NBUF)]
)


@functools.partial(
    pl.kernel,
    out_type=jax.ShapeDtypeStruct((_M_TAIL, _N), jnp.float32),
    mesh=_mesh,
    scratch_types=_scratch,
    compiler_params=pltpu.CompilerParams(use_tc_tiling_on_sc=True),
)
def _sc_add(l_hbm, r_hbm, o_hbm, *refs):
    lbuf = refs[0:_NBUF]
    rbuf = refs[_NBUF:2 * _NBUF]
    obuf = refs[2 * _NBUF:3 * _NBUF]
    sems = refs[3 * _NBUF:]
    lsem = sems[0:_NBUF]
    rsem = sems[_NBUF:2 * _NBUF]
    osem = sems[2 * _NBUF:3 * _NBUF]

    wid = lax.axis_index("s") * 2 + lax.axis_index("c")
    irow = _M_HEAD + wid * _ROWS_W
    orow = wid * _ROWS_W

    def islice(ci):
        return pl.ds(irow + ci * _CROWS, _CROWS)

    def oslice(ci):
        return pl.ds(orow + ci * _CROWS, _CROWS)

    for p in range(_NBUF - 1):
        pltpu.async_copy(l_hbm.at[islice(p)], lbuf[p], lsem[p])
        pltpu.async_copy(r_hbm.at[islice(p)], rbuf[p], rsem[p])

    @pl.loop(0, _NCHUNK, step=_NBUF)
    def chunk_group(ci0):
        for b in range(_NBUF):
            ci = ci0 + b
            pb = (b + _NBUF - 1) % _NBUF

            @pl.when(ci + _NBUF - 1 < _NCHUNK)
            def _start_ahead():
                sl = islice(ci + _NBUF - 1)
                pltpu.async_copy(l_hbm.at[sl], lbuf[pb], lsem[pb])
                pltpu.async_copy(r_hbm.at[sl], rbuf[pb], rsem[pb])

            pltpu.make_async_copy(l_hbm.at[islice(ci)], lbuf[b], lsem[b]).wait()
            pltpu.make_async_copy(r_hbm.at[islice(ci)], rbuf[b], rsem[b]).wait()

            @pl.when(ci >= _NBUF)
            def _drain_prev_out():
                pltpu.make_async_copy(
                    obuf[b], o_hbm.at[oslice(ci)], osem[b]).wait()

            lb, rb_, ob = lbuf[b], rbuf[b], obuf[b]

            def vbody(r):
                for g in range(_GROUPS):
                    sl = pl.ds(g * _LANES, _LANES)
                    ob[r, sl] = lb[r, sl] + rb_[r, sl]

            plsc.parallel_loop(0, _CROWS, 1, unroll=2)(vbody)

            pltpu.async_copy(obuf[b], o_hbm.at[oslice(ci)], osem[b])

    for b in range(_NBUF):
        pltpu.make_async_copy(obuf[b], o_hbm.at[oslice(b)], osem[b]).wait()


_BM = 4096


def _tc_add_body(l_ref, r_ref, o_ref):
    o_ref[...] = l_ref[...] + r_ref[...]


def _tc_head(left, right):
    return pl.pallas_call(
        _tc_add_body,
        grid=(_M_HEAD // _BM,),
        in_specs=[
            pl.BlockSpec((_BM, _N), lambda i: (i, 0)),
            pl.BlockSpec((_BM, _N), lambda i: (i, 0)),
        ],
        out_specs=pl.BlockSpec((_BM, _N), lambda i: (i, 0)),
        out_shape=jax.ShapeDtypeStruct((_M, _N), jnp.float32),
    )(left, right)


def _merge_body(full_ref, tail_ref, o_ref):
    o_ref[...] = tail_ref[...]


def _merge(full, sc_tail):
    nh = _M_HEAD // _BM
    return pl.pallas_call(
        _merge_body,
        grid=(_M_TAIL // _BM,),
        in_specs=[
            pl.BlockSpec((_BM, _N), lambda i, nh=nh: (i + nh, 0)),
            pl.BlockSpec((_BM, _N), lambda i: (i, 0)),
        ],
        out_specs=pl.BlockSpec((_BM, _N), lambda i, nh=nh: (i + nh, 0)),
        out_shape=jax.ShapeDtypeStruct((_M, _N), jnp.float32),
        input_output_aliases={0: 0},
    )(full, sc_tail)


def kernel(left, right):
    return _sc_add(left, right)


# speedup vs baseline: 1.1318x; 1.0024x over previous
import functools

import jax
import jax.numpy as jnp
from jax import lax
from jax.experimental import pallas as pl
from jax.experimental.pallas import tpu as pltpu
from jax.experimental.pallas import tpu_sc as plsc

_M, _N = 36864, 384
_M_TAIL = 36864
_M_HEAD = _M - _M_TAIL

_NW = 32
_ROWS_W = _M_TAIL // _NW
_NBUF = 4
_CROWS = 24
_NCHUNK = _ROWS_W // _CROWS
_LANES = 16
_GROUPS = _N // _LANES

_mesh = plsc.VectorSubcoreMesh(core_axis_name="c", subcore_axis_name="s")

_scratch = (
    [pltpu.VMEM((_CROWS, _N), jnp.float32) for _ in range(3 * _NBUF)]
    + [pltpu.SemaphoreType.DMA for _ in range(3 * _NBUF)]
)


@functools.partial(
    pl.kernel,
    out_type=jax.ShapeDtypeStruct((_M_TAIL, _N), jnp.float32),
    mesh=_mesh,
    scratch_types=_scratch,
    compiler_params=pltpu.CompilerParams(use_tc_tiling_on_sc=True),
)
def _sc_add(l_hbm, r_hbm, o_hbm, *refs):
    lbuf = refs[0:_NBUF]
    rbuf = refs[_NBUF:2 * _NBUF]
    obuf = refs[2 * _NBUF:3 * _NBUF]
    sems = refs[3 * _NBUF:]
    lsem = sems[0:_NBUF]
    rsem = sems[_NBUF:2 * _NBUF]
    osem = sems[2 * _NBUF:3 * _NBUF]

    wid = lax.axis_index("s") * 2 + lax.axis_index("c")
    irow = _M_HEAD + wid * _ROWS_W
    orow = wid * _ROWS_W

    def islice(ci):
        return pl.ds(irow + ci * _CROWS, _CROWS)

    def oslice(ci):
        return pl.ds(orow + ci * _CROWS, _CROWS)

    for p in range(_NBUF - 1):
        pltpu.async_copy(l_hbm.at[islice(p)], lbuf[p], lsem[p])
        pltpu.async_copy(r_hbm.at[islice(p)], rbuf[p], rsem[p])

    @pl.loop(0, _NCHUNK, step=_NBUF)
    def chunk_group(ci0):
        for b in range(_NBUF):
            ci = ci0 + b
            pb = (b + _NBUF - 1) % _NBUF

            @pl.when(ci + _NBUF - 1 < _NCHUNK)
            def _start_ahead():
                sl = islice(ci + _NBUF - 1)
                pltpu.async_copy(l_hbm.at[sl], lbuf[pb], lsem[pb])
                pltpu.async_copy(r_hbm.at[sl], rbuf[pb], rsem[pb])

            pltpu.make_async_copy(l_hbm.at[islice(ci)], lbuf[b], lsem[b]).wait()
            pltpu.make_async_copy(r_hbm.at[islice(ci)], rbuf[b], rsem[b]).wait()

            @pl.when(ci >= _NBUF)
            def _drain_prev_out():
                pltpu.make_async_copy(
                    obuf[b], o_hbm.at[oslice(ci)], osem[b]).wait()

            lb, rb_, ob = lbuf[b], rbuf[b], obuf[b]

            def vbody(r):
                for g in range(_GROUPS):
                    sl = pl.ds(g * _LANES, _LANES)
                    ob[r, sl] = lb[r, sl] + rb_[r, sl]

            plsc.parallel_loop(0, _CROWS, 1, unroll=2)(vbody)

            pltpu.async_copy(obuf[b], o_hbm.at[oslice(ci)], osem[b])

    for b in range(_NBUF):
        pltpu.make_async_copy(obuf[b], o_hbm.at[oslice(b)], osem[b]).wait()


_BM = 4096


def _tc_add_body(l_ref, r_ref, o_ref):
    o_ref[...] = l_ref[...] + r_ref[...]


def _tc_head(left, right):
    return pl.pallas_call(
        _tc_add_body,
        grid=(_M_HEAD // _BM,),
        in_specs=[
            pl.BlockSpec((_BM, _N), lambda i: (i, 0)),
            pl.BlockSpec((_BM, _N), lambda i: (i, 0)),
        ],
        out_specs=pl.BlockSpec((_BM, _N), lambda i: (i, 0)),
        out_shape=jax.ShapeDtypeStruct((_M, _N), jnp.float32),
    )(left, right)


def _merge_body(full_ref, tail_ref, o_ref):
    o_ref[...] = tail_ref[...]


def _merge(full, sc_tail):
    nh = _M_HEAD // _BM
    return pl.pallas_call(
        _merge_body,
        grid=(_M_TAIL // _BM,),
        in_specs=[
            pl.BlockSpec((_BM, _N), lambda i, nh=nh: (i + nh, 0)),
            pl.BlockSpec((_BM, _N), lambda i: (i, 0)),
        ],
        out_specs=pl.BlockSpec((_BM, _N), lambda i, nh=nh: (i + nh, 0)),
        out_shape=jax.ShapeDtypeStruct((_M, _N), jnp.float32),
        input_output_aliases={0: 0},
    )(full, sc_tail)


def kernel(left, right):
    return _sc_add(left, right)


# speedup vs baseline: 1.1392x; 1.0066x over previous
import functools

import jax
import jax.numpy as jnp
from jax import lax
from jax.experimental import pallas as pl
from jax.experimental.pallas import tpu as pltpu
from jax.experimental.pallas import tpu_sc as plsc

_M, _N = 36864, 384
_M_TAIL = 18432
_M_HEAD = _M - _M_TAIL

_NW = 32
_ROWS_W = _M_TAIL // _NW
_NBUF = 4
_CROWS = 24
_NCHUNK = _ROWS_W // _CROWS
_LANES = 16
_GROUPS = _N // _LANES

_mesh = plsc.VectorSubcoreMesh(core_axis_name="c", subcore_axis_name="s")

_scratch = (
    [pltpu.VMEM((_CROWS, _N), jnp.float32) for _ in range(3 * _NBUF)]
    + [pltpu.SemaphoreType.DMA for _ in range(3 * _NBUF)]
)


@functools.partial(
    pl.kernel,
    out_type=jax.ShapeDtypeStruct((_M, _N), jnp.float32),
    mesh=_mesh,
    scratch_types=_scratch,
    compiler_params=pltpu.CompilerParams(use_tc_tiling_on_sc=True),
)
def _sc_add(l_hbm, r_hbm, o_hbm, *refs):
    lbuf = refs[0:_NBUF]
    rbuf = refs[_NBUF:2 * _NBUF]
    obuf = refs[2 * _NBUF:3 * _NBUF]
    sems = refs[3 * _NBUF:]
    lsem = sems[0:_NBUF]
    rsem = sems[_NBUF:2 * _NBUF]
    osem = sems[2 * _NBUF:3 * _NBUF]

    wid = lax.axis_index("s") * 2 + lax.axis_index("c")
    irow = _M_HEAD + wid * _ROWS_W
    orow = irow

    def islice(ci):
        return pl.ds(irow + ci * _CROWS, _CROWS)

    def oslice(ci):
        return pl.ds(orow + ci * _CROWS, _CROWS)

    for p in range(_NBUF - 1):
        pltpu.async_copy(l_hbm.at[islice(p)], lbuf[p], lsem[p])
        pltpu.async_copy(r_hbm.at[islice(p)], rbuf[p], rsem[p])

    @pl.loop(0, _NCHUNK, step=_NBUF)
    def chunk_group(ci0):
        for b in range(_NBUF):
            ci = ci0 + b
            pb = (b + _NBUF - 1) % _NBUF

            @pl.when(ci + _NBUF - 1 < _NCHUNK)
            def _start_ahead():
                sl = islice(ci + _NBUF - 1)
                pltpu.async_copy(l_hbm.at[sl], lbuf[pb], lsem[pb])
                pltpu.async_copy(r_hbm.at[sl], rbuf[pb], rsem[pb])

            pltpu.make_async_copy(l_hbm.at[islice(ci)], lbuf[b], lsem[b]).wait()
            pltpu.make_async_copy(r_hbm.at[islice(ci)], rbuf[b], rsem[b]).wait()

            @pl.when(ci >= _NBUF)
            def _drain_prev_out():
                pltpu.make_async_copy(
                    obuf[b], o_hbm.at[oslice(ci)], osem[b]).wait()

            lb, rb_, ob = lbuf[b], rbuf[b], obuf[b]

            def vbody(r):
                for g in range(_GROUPS):
                    sl = pl.ds(g * _LANES, _LANES)
                    ob[r, sl] = lb[r, sl] + rb_[r, sl]

            plsc.parallel_loop(0, _CROWS, 1, unroll=2)(vbody)

            pltpu.async_copy(obuf[b], o_hbm.at[oslice(ci)], osem[b])

    for b in range(_NBUF):
        pltpu.make_async_copy(obuf[b], o_hbm.at[oslice(b)], osem[b]).wait()


_BM = 2048


def _tc_add_body(l_ref, r_ref, o_ref):
    o_ref[...] = l_ref[...] + r_ref[...]


def _tc_head_into_body(l_ref, r_ref, full_ref, o_ref):
    o_ref[...] = l_ref[...] + r_ref[...]


def _tc_head_into(left, right, sc_full):
    return pl.pallas_call(
        _tc_head_into_body,
        grid=(_M_HEAD // _BM,),
        in_specs=[
            pl.BlockSpec((_BM, _N), lambda i: (i, 0)),
            pl.BlockSpec((_BM, _N), lambda i: (i, 0)),
            pl.BlockSpec(memory_space=pltpu.MemorySpace.HBM),
        ],
        out_specs=pl.BlockSpec((_BM, _N), lambda i: (i, 0)),
        out_shape=jax.ShapeDtypeStruct((_M, _N), jnp.float32),
        input_output_aliases={2: 0},
    )(left, right, sc_full)


def _tc_head(left, right):
    return pl.pallas_call(
        _tc_add_body,
        grid=(_M_HEAD // _BM,),
        in_specs=[
            pl.BlockSpec((_BM, _N), lambda i: (i, 0)),
            pl.BlockSpec((_BM, _N), lambda i: (i, 0)),
        ],
        out_specs=pl.BlockSpec((_BM, _N), lambda i: (i, 0)),
        out_shape=jax.ShapeDtypeStruct((_M, _N), jnp.float32),
    )(left, right)


def _merge_body(full_ref, tail_ref, o_ref):
    o_ref[...] = tail_ref[...]


def _merge(full, sc_tail):
    nh = _M_HEAD // _BM
    return pl.pallas_call(
        _merge_body,
        grid=(_M_TAIL // _BM,),
        in_specs=[
            pl.BlockSpec((_BM, _N), lambda i, nh=nh: (i + nh, 0)),
            pl.BlockSpec((_BM, _N), lambda i: (i, 0)),
        ],
        out_specs=pl.BlockSpec((_BM, _N), lambda i, nh=nh: (i + nh, 0)),
        out_shape=jax.ShapeDtypeStruct((_M, _N), jnp.float32),
        input_output_aliases={0: 0},
    )(full, sc_tail)


def kernel(left, right):
    sc_full = _sc_add(left, right)
    return _tc_head_into(left, right, sc_full)


# speedup vs baseline: 1.1410x; 1.0015x over previous
import functools

import jax
import jax.numpy as jnp
from jax import lax
from jax.experimental import pallas as pl
from jax.experimental.pallas import tpu as pltpu
from jax.experimental.pallas import tpu_sc as plsc

_M, _N = 36864, 384
_M_TAIL = 18432
_M_HEAD = _M - _M_TAIL

_NW = 32
_ROWS_W = _M_TAIL // _NW
_NBUF = 4
_CROWS = 24
_NCHUNK = _ROWS_W // _CROWS
_LANES = 16
_GROUPS = _N // _LANES

_mesh = plsc.VectorSubcoreMesh(core_axis_name="c", subcore_axis_name="s")

_scratch = (
    [pltpu.VMEM((_CROWS, _N), jnp.float32) for _ in range(3 * _NBUF)]
    + [pltpu.SemaphoreType.DMA for _ in range(3 * _NBUF)]
)


@functools.partial(
    pl.kernel,
    out_type=jax.ShapeDtypeStruct((_M, _N), jnp.float32),
    mesh=_mesh,
    scratch_types=_scratch,
    compiler_params=pltpu.CompilerParams(use_tc_tiling_on_sc=True),
)
def _sc_add(l_hbm, r_hbm, o_hbm, *refs):
    lbuf = refs[0:_NBUF]
    rbuf = refs[_NBUF:2 * _NBUF]
    obuf = refs[2 * _NBUF:3 * _NBUF]
    sems = refs[3 * _NBUF:]
    lsem = sems[0:_NBUF]
    rsem = sems[_NBUF:2 * _NBUF]
    osem = sems[2 * _NBUF:3 * _NBUF]

    wid = lax.axis_index("s") * 2 + lax.axis_index("c")
    row0 = _M_HEAD + wid * _ROWS_W

    def rslice(ci):
        return pl.ds(row0 + ci * _CROWS, _CROWS)

    for p in range(_NBUF - 1):
        pltpu.async_copy(l_hbm.at[rslice(p)], lbuf[p], lsem[p])
        pltpu.async_copy(r_hbm.at[rslice(p)], rbuf[p], rsem[p])

    @pl.loop(0, _NCHUNK, step=_NBUF)
    def chunk_group(ci0):
        for b in range(_NBUF):
            ci = ci0 + b
            pb = (b + _NBUF - 1) % _NBUF

            @pl.when(ci + _NBUF - 1 < _NCHUNK)
            def _start_ahead():
                sl = rslice(ci + _NBUF - 1)
                pltpu.async_copy(l_hbm.at[sl], lbuf[pb], lsem[pb])
                pltpu.async_copy(r_hbm.at[sl], rbuf[pb], rsem[pb])

            pltpu.make_async_copy(l_hbm.at[rslice(ci)], lbuf[b], lsem[b]).wait()
            pltpu.make_async_copy(r_hbm.at[rslice(ci)], rbuf[b], rsem[b]).wait()

            @pl.when(ci >= _NBUF)
            def _drain_prev_out():
                pltpu.make_async_copy(
                    obuf[b], o_hbm.at[rslice(ci)], osem[b]).wait()

            lb, rb_, ob = lbuf[b], rbuf[b], obuf[b]

            def vbody(r):
                for g in range(_GROUPS):
                    sl = pl.ds(g * _LANES, _LANES)
                    ob[r, sl] = lb[r, sl] + rb_[r, sl]

            plsc.parallel_loop(0, _CROWS, 1, unroll=2)(vbody)

            pltpu.async_copy(obuf[b], o_hbm.at[rslice(ci)], osem[b])

    for b in range(_NBUF):
        pltpu.make_async_copy(obuf[b], o_hbm.at[rslice(b)], osem[b]).wait()


_BM = 2048


def _tc_head_into_body(l_ref, r_ref, full_ref, o_ref):
    o_ref[...] = l_ref[...] + r_ref[...]


def _tc_head_into(left, right, sc_full):
    return pl.pallas_call(
        _tc_head_into_body,
        grid=(_M_HEAD // _BM,),
        in_specs=[
            pl.BlockSpec((_BM, _N), lambda i: (i, 0)),
            pl.BlockSpec((_BM, _N), lambda i: (i, 0)),
            pl.BlockSpec(memory_space=pltpu.MemorySpace.HBM),
        ],
        out_specs=pl.BlockSpec((_BM, _N), lambda i: (i, 0)),
        out_shape=jax.ShapeDtypeStruct((_M, _N), jnp.float32),
        input_output_aliases={2: 0},
    )(left, right, sc_full)


def kernel(left, right):
    sc_full = _sc_add(left, right)
    return _tc_head_into(left, right, sc_full)
